# baseline (speedup 1.0000x reference)
import functools

import jax
import jax.numpy as jnp
from jax import lax
from jax.experimental import pallas as pl
from jax.experimental.pallas import tpu as pltpu

_BF = jnp.bfloat16
_F32 = jnp.float32


def _dot(a, b):
    return jnp.dot(a, b, preferred_element_type=_F32)


def _taps3(arr, axis, size, stride):
    if stride == 1:
        return [lax.slice_in_dim(arr, d, d + size, axis=axis) for d in range(3)]
    ns = arr.shape[axis] // 2
    shp = arr.shape[:axis] + (ns, 2) + arr.shape[axis + 1:]
    r = arr.reshape(shp)
    ev = lax.index_in_dim(r, 0, axis=axis + 1, keepdims=False)
    od = lax.index_in_dim(r, 1, axis=axis + 1, keepdims=False)
    return [lax.slice_in_dim(ev, 0, size, axis=axis),
            lax.slice_in_dim(od, 0, size, axis=axis),
            lax.slice_in_dim(ev, 1, size + 1, axis=axis)]


def _sub2(arr, axis):
    ns = arr.shape[axis] // 2
    shp = arr.shape[:axis] + (ns, 2) + arr.shape[axis + 1:]
    return lax.index_in_dim(arr.reshape(shp), 0, axis=axis + 1, keepdims=False)


def _shift_w(t, d):
    n, h, w, c = t.shape
    z = jnp.zeros((n, h, 1, c), t.dtype)
    if d > 0:
        return jnp.concatenate([z, t[:, :, :w - 1, :]], axis=2)
    return jnp.concatenate([t[:, :, 1:, :], z], axis=2)


def _evod(t, which):
    n, h, w, c = t.shape
    r = t.reshape(n, h, w // 2, 2, c)
    return lax.index_in_dim(r, which, axis=3, keepdims=False)


def _bneck_body(*refs, stride, has_cd):
    if has_cd:
        (x_ref, w1, s1, c1, w2, s2, c2, w3, s3, c3, wd, sd, cd, o_ref) = refs
    else:
        (x_ref, w1, s1, c1, w2, s2, c2, w3, s3, c3, o_ref) = refs
    ips, H, W, Cin = x_ref.shape
    _, HO, WO, Cout = o_ref.shape
    N1 = w1.shape[1]
    x = x_ref[...]

    h1 = jnp.maximum(_dot(x.reshape(ips * H * W, Cin), w1[...]) * s1[...]
                     + c1[...], 0.0)
    h1 = h1.astype(_BF).reshape(ips, H, W, N1)

    hp = jnp.pad(h1, ((0, 0), (1, 1), (0, 0), (0, 0)))
    rows = [r.reshape(ips * HO * W, N1) for r in _taps3(hp, 1, HO, stride)]
    m2 = ips * HO * WO
    t = []
    for dj in range(3):
        acc = _dot(rows[0], w2[dj])
        acc = acc + _dot(rows[1], w2[3 + dj])
        acc = acc + _dot(rows[2], w2[6 + dj])
        t.append(acc.reshape(ips, HO, W, -1))
    if stride == 1:
        y = t[1] + _shift_w(t[0], +1) + _shift_w(t[2], -1)
    else:
        y = _evod(t[1], 0) + _evod(t[2], 1) + _shift_w(_evod(t[0], 1), +1)
    h2 = jnp.maximum(y.reshape(m2, -1) * s2[...] + c2[...], 0.0).astype(_BF)

    h3 = _dot(h2, w3[...]) * s3[...] + c3[...]
    if has_cd:
        xd = x if stride == 1 else _sub2(_sub2(x, 1), 2)
        res = _dot(xd.reshape(m2, Cin), wd[...]) * sd[...] + cd[...]
    else:
        res = x.reshape(m2, Cout).astype(_F32)
    o_ref[...] = jnp.maximum(h3 + res, 0.0).astype(_BF).reshape(
        ips, HO, WO, Cout)


def _bottleneck(x, p1, p2, p3, pd, stride):
    n, H, W, Cin = x.shape
    HO = (H - 1) // stride + 1
    WO = (W - 1) // stride + 1
    w1, s1, c1 = p1
    b2, s2, c2 = p2
    w3, s3, c3 = p3
    N1 = w1.shape[1]
    N2 = b2.shape[1]
    Cout = w3.shape[1]
    w2 = b2.reshape(9, N1, N2)
    ips = {56: 1, 28: 2, 14: 4}.get(H, 4)
    z4 = lambda i: (i, 0, 0, 0)
    zw = lambda i: (0, 0)
    zw3 = lambda i: (0, 0, 0)
    in_specs = [
        pl.BlockSpec((ips, H, W, Cin), z4),
        pl.BlockSpec(w1.shape, zw), pl.BlockSpec(s1.shape, zw),
        pl.BlockSpec(c1.shape, zw),
        pl.BlockSpec(w2.shape, zw3), pl.BlockSpec(s2.shape, zw),
        pl.BlockSpec(c2.shape, zw),
        pl.BlockSpec(w3.shape, zw), pl.BlockSpec(s3.shape, zw),
        pl.BlockSpec(c3.shape, zw),
    ]
    args = [x, w1, s1, c1, w2, s2, c2, w3, s3, c3]
    if pd is not None:
        wd, sd, cd = pd
        in_specs += [pl.BlockSpec(wd.shape, zw), pl.BlockSpec(sd.shape, zw),
                     pl.BlockSpec(cd.shape, zw)]
        args += [wd, sd, cd]
    return pl.pallas_call(
        functools.partial(_bneck_body, stride=stride, has_cd=pd is not None),
        out_shape=jax.ShapeDtypeStruct((n, HO, WO, Cout), _BF),
        grid=(n // ips,),
        in_specs=in_specs,
        out_specs=pl.BlockSpec((ips, HO, WO, Cout), z4),
        compiler_params=pltpu.CompilerParams(dimension_semantics=("parallel",)),
    )(*args)


def _conv1_pool_body(x_ref, a_ref, o_ref):
    _, HO, WO, N = o_ref.shape
    Hc, Wc = 2 * HO, 2 * WO
    Wt = 2 * Wc
    planes = []
    for i in range(7):
        for j in range(7):
            for ch in range(3):
                base = x_ref[0, ch]
                rr = base.reshape(115, 2, 230)
                planes.append(rr[i // 2:i // 2 + Hc, i % 2,
                                 j:j + Wt])
    planes.append(jnp.ones((Hc, Wt), _BF))
    bm = jnp.stack(planes, axis=0).reshape(len(planes), Hc * Wt)
    yc = jnp.maximum(_dot(a_ref[...], bm), 0.0).astype(_BF)
    eye = jnp.eye(N, dtype=_BF)
    yt = lax.dot_general(yc, eye, (((0,), (0,)), ((), ())),
                         preferred_element_type=_F32)
    y = lax.index_in_dim(yt.reshape(Hc, Wc, 2, N), 0, axis=2,
                         keepdims=False)
    yp = jnp.pad(y, ((1, 1), (1, 1), (0, 0)))
    r = _taps3(yp, 0, HO, 2)
    ym = jnp.maximum(jnp.maximum(r[0], r[1]), r[2])
    cs = _taps3(ym, 1, WO, 2)
    o = jnp.maximum(jnp.maximum(cs[0], cs[1]), cs[2])
    o_ref[...] = o.astype(_BF)[None]


def _conv1_pool(x_nchw, b, s, c):
    n = x_nchw.shape[0]
    xp = jnp.pad(x_nchw.astype(_BF), ((0, 0), (0, 0), (3, 3), (3, 3)))
    a = jnp.concatenate(
        [b[:147].astype(_F32).T * s.reshape(-1, 1), c.reshape(-1, 1)],
        axis=1).astype(_BF)
    HO = 56
    return pl.pallas_call(
        _conv1_pool_body,
        out_shape=jax.ShapeDtypeStruct((n, HO, HO, a.shape[0]), _BF),
        grid=(n,),
        in_specs=[
            pl.BlockSpec((1, 3, 230, 230), lambda i: (i, 0, 0, 0)),
            pl.BlockSpec(a.shape, lambda i: (0, 0)),
        ],
        out_specs=pl.BlockSpec((1, HO, HO, a.shape[0]), lambda i: (i, 0, 0, 0)),
        compiler_params=pltpu.CompilerParams(dimension_semantics=("parallel",)),
    )(xp, a)


def _nchw_body(x_ref, o_ref):
    o_ref[...] = jnp.transpose(x_ref[0], (2, 0, 1)).astype(_F32)[None]


def _to_nchw(x):
    n, h, w, ch = x.shape
    cb = ch // 128
    return pl.pallas_call(
        _nchw_body,
        out_shape=jax.ShapeDtypeStruct((n, ch, h, w), _F32),
        grid=(n, cb),
        in_specs=[pl.BlockSpec((1, h, w, 128), lambda i, j: (i, 0, 0, j))],
        out_specs=pl.BlockSpec((1, 128, h, w), lambda i, j: (i, j, 0, 0)),
        compiler_params=pltpu.CompilerParams(
            dimension_semantics=("parallel", "parallel")),
    )(x)


def _gap_body(x_ref, o_ref):
    o_ref[...] = jnp.mean(x_ref[...].astype(_F32), axis=1)


def _gap(x):
    n, h, w, ch = x.shape
    xr = x.reshape(n, h * w, ch)
    tc = ch // 2
    out = pl.pallas_call(
        _gap_body,
        out_shape=jax.ShapeDtypeStruct((n, ch), _F32),
        grid=(2,),
        in_specs=[pl.BlockSpec((n, h * w, tc), lambda j: (0, 0, j))],
        out_specs=pl.BlockSpec((n, tc), lambda j: (0, j)),
        compiler_params=pltpu.CompilerParams(dimension_semantics=("parallel",)),
    )(xr)
    return out.reshape(n, ch, 1, 1)


_LAYERS = ((64, 3, 1), (128, 4, 2), (256, 6, 2), (512, 3, 2))


def kernel(x, conv1_b, conv1_s, conv1_c, L0B0_c1_b, L0B0_c1_s, L0B0_c1_c, L0B0_c2_b, L0B0_c2_s, L0B0_c2_c, L0B0_c3_b, L0B0_c3_s, L0B0_c3_c, L0B0_cd_b, L0B0_cd_s, L0B0_cd_c, L0B1_c1_b, L0B1_c1_s, L0B1_c1_c, L0B1_c2_b, L0B1_c2_s, L0B1_c2_c, L0B1_c3_b, L0B1_c3_s, L0B1_c3_c, L0B2_c1_b, L0B2_c1_s, L0B2_c1_c, L0B2_c2_b, L0B2_c2_s, L0B2_c2_c, L0B2_c3_b, L0B2_c3_s, L0B2_c3_c, L1B0_c1_b, L1B0_c1_s, L1B0_c1_c, L1B0_c2_b, L1B0_c2_s, L1B0_c2_c, L1B0_c3_b, L1B0_c3_s, L1B0_c3_c, L1B0_cd_b, L1B0_cd_s, L1B0_cd_c, L1B1_c1_b, L1B1_c1_s, L1B1_c1_c, L1B1_c2_b, L1B1_c2_s, L1B1_c2_c, L1B1_c3_b, L1B1_c3_s, L1B1_c3_c, L1B2_c1_b, L1B2_c1_s, L1B2_c1_c, L1B2_c2_b, L1B2_c2_s, L1B2_c2_c, L1B2_c3_b, L1B2_c3_s, L1B2_c3_c, L1B3_c1_b, L1B3_c1_s, L1B3_c1_c, L1B3_c2_b, L1B3_c2_s, L1B3_c2_c, L1B3_c3_b, L1B3_c3_s, L1B3_c3_c, L2B0_c1_b, L2B0_c1_s, L2B0_c1_c, L2B0_c2_b, L2B0_c2_s, L2B0_c2_c, L2B0_c3_b, L2B0_c3_s, L2B0_c3_c, L2B0_cd_b, L2B0_cd_s, L2B0_cd_c, L2B1_c1_b, L2B1_c1_s, L2B1_c1_c, L2B1_c2_b, L2B1_c2_s, L2B1_c2_c, L2B1_c3_b, L2B1_c3_s, L2B1_c3_c, L2B2_c1_b, L2B2_c1_s, L2B2_c1_c, L2B2_c2_b, L2B2_c2_s, L2B2_c2_c, L2B2_c3_b, L2B2_c3_s, L2B2_c3_c, L2B3_c1_b, L2B3_c1_s, L2B3_c1_c, L2B3_c2_b, L2B3_c2_s, L2B3_c2_c, L2B3_c3_b, L2B3_c3_s, L2B3_c3_c, L2B4_c1_b, L2B4_c1_s, L2B4_c1_c, L2B4_c2_b, L2B4_c2_s, L2B4_c2_c, L2B4_c3_b, L2B4_c3_s, L2B4_c3_c, L2B5_c1_b, L2B5_c1_s, L2B5_c1_c, L2B5_c2_b, L2B5_c2_s, L2B5_c2_c, L2B5_c3_b, L2B5_c3_s, L2B5_c3_c, L3B0_c1_b, L3B0_c1_s, L3B0_c1_c, L3B0_c2_b, L3B0_c2_s, L3B0_c2_c, L3B0_c3_b, L3B0_c3_s, L3B0_c3_c, L3B0_cd_b, L3B0_cd_s, L3B0_cd_c, L3B1_c1_b, L3B1_c1_s, L3B1_c1_c, L3B1_c2_b, L3B1_c2_s, L3B1_c2_c, L3B1_c3_b, L3B1_c3_s, L3B1_c3_c, L3B2_c1_b, L3B2_c1_s, L3B2_c1_c, L3B2_c2_b, L3B2_c2_s, L3B2_c2_c, L3B2_c3_b, L3B2_c3_s, L3B2_c3_c):
    _loc = dict(locals())
    act = _conv1_pool(x, conv1_b, conv1_s, conv1_c)
    outs = []
    for li, (_, blocks, stride0) in enumerate(_LAYERS):
        for bi in range(blocks):
            def g(part, t):
                return _loc[f"L{li}B{bi}_{part}_{t}"]
            p1 = (g("c1", "b"), g("c1", "s"), g("c1", "c"))
            p2 = (g("c2", "b"), g("c2", "s"), g("c2", "c"))
            p3 = (g("c3", "b"), g("c3", "s"), g("c3", "c"))
            pd = ((g("cd", "b"), g("cd", "s"), g("cd", "c"))
                  if f"L{li}B{bi}_cd_b" in _loc else None)
            act = _bottleneck(act, p1, p2, p3, pd, stride0 if bi == 0 else 1)
        outs.append(_to_nchw(act))
    outs.append(_gap(act))
    return tuple(outs)


# speedup vs baseline: 1.5148x; 1.5148x over previous
import functools

import jax
import jax.numpy as jnp
from jax import lax
from jax.experimental import pallas as pl
from jax.experimental.pallas import tpu as pltpu

_BF = jnp.bfloat16
_F32 = jnp.float32


def _dot(a, b):
    return jnp.dot(a, b, preferred_element_type=_F32)


def _taps3(arr, axis, size, stride):
    if stride == 1:
        return [lax.slice_in_dim(arr, d, d + size, axis=axis) for d in range(3)]
    ns = arr.shape[axis] // 2
    shp = arr.shape[:axis] + (ns, 2) + arr.shape[axis + 1:]
    r = arr.reshape(shp)
    ev = lax.index_in_dim(r, 0, axis=axis + 1, keepdims=False)
    od = lax.index_in_dim(r, 1, axis=axis + 1, keepdims=False)
    return [lax.slice_in_dim(ev, 0, size, axis=axis),
            lax.slice_in_dim(od, 0, size, axis=axis),
            lax.slice_in_dim(ev, 1, size + 1, axis=axis)]


def _sub2(arr, axis):
    ns = arr.shape[axis] // 2
    shp = arr.shape[:axis] + (ns, 2) + arr.shape[axis + 1:]
    return lax.index_in_dim(arr.reshape(shp), 0, axis=axis + 1, keepdims=False)


def _shift_w(t, d):
    n, h, w, c = t.shape
    z = jnp.zeros((n, h, 1, c), t.dtype)
    if d > 0:
        return jnp.concatenate([z, t[:, :, :w - 1, :]], axis=2)
    return jnp.concatenate([t[:, :, 1:, :], z], axis=2)


def _evod(t, which):
    n, h, w, c = t.shape
    r = t.reshape(n, h, w // 2, 2, c)
    return lax.index_in_dim(r, which, axis=3, keepdims=False)


def _bneck_math(x, w1, s1, c1, w2, s2, c2, w3, s3, c3, wd, sd, cd, stride):
    has_cd = wd is not None
    ips, H, W, Cin = x.shape
    HO = (H - 1) // stride + 1
    WO = (W - 1) // stride + 1
    N1 = w1.shape[1]
    Cout = w3.shape[1]

    h1 = jnp.maximum(_dot(x.reshape(ips * H * W, Cin), w1[...]) * s1[...]
                     + c1[...], 0.0)
    h1 = h1.astype(_BF).reshape(ips, H, W, N1)

    hp = jnp.pad(h1, ((0, 0), (1, 1), (0, 0), (0, 0)))
    rows = [r.reshape(ips * HO * W, N1) for r in _taps3(hp, 1, HO, stride)]
    m2 = ips * HO * WO
    t = []
    for dj in range(3):
        acc = _dot(rows[0], w2[dj])
        acc = acc + _dot(rows[1], w2[3 + dj])
        acc = acc + _dot(rows[2], w2[6 + dj])
        t.append(acc.reshape(ips, HO, W, -1))
    if stride == 1:
        y = t[1] + _shift_w(t[0], +1) + _shift_w(t[2], -1)
    else:
        y = _evod(t[1], 0) + _evod(t[2], 1) + _shift_w(_evod(t[0], 1), +1)
    h2 = jnp.maximum(y.reshape(m2, -1) * s2[...] + c2[...], 0.0).astype(_BF)

    h3 = _dot(h2, w3[...]) * s3[...] + c3[...]
    if has_cd:
        xd = x if stride == 1 else _sub2(_sub2(x, 1), 2)
        res = _dot(xd.reshape(m2, Cin), wd[...]) * sd[...] + cd[...]
    else:
        res = x.reshape(m2, Cout).astype(_F32)
    return jnp.maximum(h3 + res, 0.0).astype(_BF).reshape(ips, HO, WO, Cout)


def _stage_body(*refs, strides, cds):
    x_ref, o_ref = refs[0], refs[-1]
    x = x_ref[...]
    idx = 1
    for b, stride in enumerate(strides):
        n_p = 12 if cds[b] else 9
        p = refs[idx:idx + n_p]
        idx += n_p
        if not cds[b]:
            p = p + (None, None, None)
        x = _bneck_math(x, *p, stride)
    o_ref[...] = x


def _stage(x, plist, stride0):
    n, H, W, Cin = x.shape
    z4 = lambda i: (i, 0, 0, 0)
    zw = lambda i: (0, 0)
    zw3 = lambda i: (0, 0, 0)
    ips = {56: 1, 28: 2, 14: 4}.get(H, 4)
    args = [x]
    in_specs = [pl.BlockSpec((ips, H, W, Cin), z4)]
    strides, cds = [], []
    HO, WO, Cout = H, W, Cin
    for bi, (p1, p2, p3, pd) in enumerate(plist):
        stride = stride0 if bi == 0 else 1
        strides.append(stride)
        cds.append(pd is not None)
        HO = (HO - 1) // stride + 1
        WO = (WO - 1) // stride + 1
        Cout = p3[0].shape[1]
        N1 = p1[0].shape[1]
        w2 = p2[0].reshape(9, N1, p2[0].shape[1])
        group = [p1[0], p1[1], p1[2], w2, p2[1], p2[2], p3[0], p3[1], p3[2]]
        specs = [pl.BlockSpec(p1[0].shape, zw), pl.BlockSpec(p1[1].shape, zw),
                 pl.BlockSpec(p1[2].shape, zw),
                 pl.BlockSpec(w2.shape, zw3), pl.BlockSpec(p2[1].shape, zw),
                 pl.BlockSpec(p2[2].shape, zw),
                 pl.BlockSpec(p3[0].shape, zw), pl.BlockSpec(p3[1].shape, zw),
                 pl.BlockSpec(p3[2].shape, zw)]
        if pd is not None:
            group += [pd[0], pd[1], pd[2]]
            specs += [pl.BlockSpec(pd[0].shape, zw),
                      pl.BlockSpec(pd[1].shape, zw),
                      pl.BlockSpec(pd[2].shape, zw)]
        args += group
        in_specs += specs
    return pl.pallas_call(
        functools.partial(_stage_body, strides=tuple(strides), cds=tuple(cds)),
        out_shape=jax.ShapeDtypeStruct((n, HO, WO, Cout), _BF),
        grid=(n // ips,),
        in_specs=in_specs,
        out_specs=pl.BlockSpec((ips, HO, WO, Cout), z4),
        compiler_params=pltpu.CompilerParams(dimension_semantics=("parallel",)),
    )(*args)


def _conv1_pool_body(x_ref, a_ref, o_ref):
    _, HO, WO, N = o_ref.shape
    Hc, Wc = 2 * HO, 2 * WO
    Wt = 2 * Wc
    planes = []
    for i in range(7):
        for j in range(7):
            for ch in range(3):
                base = x_ref[0, ch]
                rr = base.reshape(115, 2, 230)
                planes.append(rr[i // 2:i // 2 + Hc, i % 2,
                                 j:j + Wt])
    planes.append(jnp.ones((Hc, Wt), _BF))
    bm = jnp.stack(planes, axis=0).reshape(len(planes), Hc * Wt)
    yc = jnp.maximum(_dot(a_ref[...], bm), 0.0).astype(_BF)
    eye = jnp.eye(N, dtype=_BF)
    yt = lax.dot_general(yc, eye, (((0,), (0,)), ((), ())),
                         preferred_element_type=_F32)
    y = lax.index_in_dim(yt.reshape(Hc, Wc, 2, N), 0, axis=2,
                         keepdims=False)
    yp = jnp.pad(y, ((1, 1), (1, 1), (0, 0)))
    r = _taps3(yp, 0, HO, 2)
    ym = jnp.maximum(jnp.maximum(r[0], r[1]), r[2])
    cs = _taps3(ym, 1, WO, 2)
    o = jnp.maximum(jnp.maximum(cs[0], cs[1]), cs[2])
    o_ref[...] = o.astype(_BF)[None]


def _conv1_pool(x_nchw, b, s, c):
    n = x_nchw.shape[0]
    xp = jnp.pad(x_nchw.astype(_BF), ((0, 0), (0, 0), (3, 3), (3, 3)))
    a = jnp.concatenate(
        [b[:147].astype(_F32).T * s.reshape(-1, 1), c.reshape(-1, 1)],
        axis=1).astype(_BF)
    HO = 56
    return pl.pallas_call(
        _conv1_pool_body,
        out_shape=jax.ShapeDtypeStruct((n, HO, HO, a.shape[0]), _BF),
        grid=(n,),
        in_specs=[
            pl.BlockSpec((1, 3, 230, 230), lambda i: (i, 0, 0, 0)),
            pl.BlockSpec(a.shape, lambda i: (0, 0)),
        ],
        out_specs=pl.BlockSpec((1, HO, HO, a.shape[0]), lambda i: (i, 0, 0, 0)),
        compiler_params=pltpu.CompilerParams(dimension_semantics=("parallel",)),
    )(xp, a)


def _gap_body(x_ref, o_ref):
    o_ref[...] = jnp.mean(x_ref[...].astype(_F32), axis=1)


def _gap(x):
    n, h, w, ch = x.shape
    xr = x.reshape(n, h * w, ch)
    tc = ch // 2
    out = pl.pallas_call(
        _gap_body,
        out_shape=jax.ShapeDtypeStruct((n, ch), _F32),
        grid=(2,),
        in_specs=[pl.BlockSpec((n, h * w, tc), lambda j: (0, 0, j))],
        out_specs=pl.BlockSpec((n, tc), lambda j: (0, j)),
        compiler_params=pltpu.CompilerParams(dimension_semantics=("parallel",)),
    )(xr)
    return out.reshape(n, ch, 1, 1)


_LAYERS = ((64, 3, 1), (128, 4, 2), (256, 6, 2), (512, 3, 2))


def kernel(x, conv1_b, conv1_s, conv1_c, L0B0_c1_b, L0B0_c1_s, L0B0_c1_c, L0B0_c2_b, L0B0_c2_s, L0B0_c2_c, L0B0_c3_b, L0B0_c3_s, L0B0_c3_c, L0B0_cd_b, L0B0_cd_s, L0B0_cd_c, L0B1_c1_b, L0B1_c1_s, L0B1_c1_c, L0B1_c2_b, L0B1_c2_s, L0B1_c2_c, L0B1_c3_b, L0B1_c3_s, L0B1_c3_c, L0B2_c1_b, L0B2_c1_s, L0B2_c1_c, L0B2_c2_b, L0B2_c2_s, L0B2_c2_c, L0B2_c3_b, L0B2_c3_s, L0B2_c3_c, L1B0_c1_b, L1B0_c1_s, L1B0_c1_c, L1B0_c2_b, L1B0_c2_s, L1B0_c2_c, L1B0_c3_b, L1B0_c3_s, L1B0_c3_c, L1B0_cd_b, L1B0_cd_s, L1B0_cd_c, L1B1_c1_b, L1B1_c1_s, L1B1_c1_c, L1B1_c2_b, L1B1_c2_s, L1B1_c2_c, L1B1_c3_b, L1B1_c3_s, L1B1_c3_c, L1B2_c1_b, L1B2_c1_s, L1B2_c1_c, L1B2_c2_b, L1B2_c2_s, L1B2_c2_c, L1B2_c3_b, L1B2_c3_s, L1B2_c3_c, L1B3_c1_b, L1B3_c1_s, L1B3_c1_c, L1B3_c2_b, L1B3_c2_s, L1B3_c2_c, L1B3_c3_b, L1B3_c3_s, L1B3_c3_c, L2B0_c1_b, L2B0_c1_s, L2B0_c1_c, L2B0_c2_b, L2B0_c2_s, L2B0_c2_c, L2B0_c3_b, L2B0_c3_s, L2B0_c3_c, L2B0_cd_b, L2B0_cd_s, L2B0_cd_c, L2B1_c1_b, L2B1_c1_s, L2B1_c1_c, L2B1_c2_b, L2B1_c2_s, L2B1_c2_c, L2B1_c3_b, L2B1_c3_s, L2B1_c3_c, L2B2_c1_b, L2B2_c1_s, L2B2_c1_c, L2B2_c2_b, L2B2_c2_s, L2B2_c2_c, L2B2_c3_b, L2B2_c3_s, L2B2_c3_c, L2B3_c1_b, L2B3_c1_s, L2B3_c1_c, L2B3_c2_b, L2B3_c2_s, L2B3_c2_c, L2B3_c3_b, L2B3_c3_s, L2B3_c3_c, L2B4_c1_b, L2B4_c1_s, L2B4_c1_c, L2B4_c2_b, L2B4_c2_s, L2B4_c2_c, L2B4_c3_b, L2B4_c3_s, L2B4_c3_c, L2B5_c1_b, L2B5_c1_s, L2B5_c1_c, L2B5_c2_b, L2B5_c2_s, L2B5_c2_c, L2B5_c3_b, L2B5_c3_s, L2B5_c3_c, L3B0_c1_b, L3B0_c1_s, L3B0_c1_c, L3B0_c2_b, L3B0_c2_s, L3B0_c2_c, L3B0_c3_b, L3B0_c3_s, L3B0_c3_c, L3B0_cd_b, L3B0_cd_s, L3B0_cd_c, L3B1_c1_b, L3B1_c1_s, L3B1_c1_c, L3B1_c2_b, L3B1_c2_s, L3B1_c2_c, L3B1_c3_b, L3B1_c3_s, L3B1_c3_c, L3B2_c1_b, L3B2_c1_s, L3B2_c1_c, L3B2_c2_b, L3B2_c2_s, L3B2_c2_c, L3B2_c3_b, L3B2_c3_s, L3B2_c3_c):
    _loc = dict(locals())
    act = _conv1_pool(x, conv1_b, conv1_s, conv1_c)
    outs = []
    for li, (_, blocks, stride0) in enumerate(_LAYERS):
        plist = []
        for bi in range(blocks):
            def g(part, t):
                return _loc[f"L{li}B{bi}_{part}_{t}"]
            p1 = (g("c1", "b"), g("c1", "s"), g("c1", "c"))
            p2 = (g("c2", "b"), g("c2", "s"), g("c2", "c"))
            p3 = (g("c3", "b"), g("c3", "s"), g("c3", "c"))
            pd = ((g("cd", "b"), g("cd", "s"), g("cd", "c"))
                  if f"L{li}B{bi}_cd_b" in _loc else None)
            plist.append((p1, p2, p3, pd))
        act = _stage(act, plist, stride0)
        outs.append(jnp.transpose(act.astype(_F32), (0, 3, 1, 2)))
    outs.append(_gap(act))
    return tuple(outs)
